# baseline (speedup 1.0000x reference)
import jax
import jax.numpy as jnp
from jax.experimental import pallas as pl
from jax.experimental.pallas import tpu as pltpu


def _dot_nk(a, w_nk):
    return jax.lax.dot_general(
        a, w_nk, (((1,), (1,)), ((), ())),
        preferred_element_type=jnp.float32)


def _mlstm_kernel(x_ref, h_ref, d0_ref, cell_ref, wi_ref, wo_ref, wc_ref,
                  wf_ref, wout_ref, bi_ref, bo_ref, bc_ref, bf_ref, bout_ref,
                  out_ref, hn_ref, hc_ref, d_ref):
    n_in = x_ref.shape[1]
    k = cell_ref.shape[0]
    x = x_ref[...]
    h = h_ref[...]

    f_pre = (_dot_nk(x, wf_ref[:, 0:n_in])
             + _dot_nk(h, wf_ref[:, n_in:n_in + h.shape[1]])
             + _dot_nk(d0_ref[...], wf_ref[:, n_in + h.shape[1]:])
             + bf_ref[...])
    d = jax.nn.sigmoid(f_pre) * 0.5
    d_ref[...] = d
    ci = (jnp.tanh(_dot_nk(x, wc_ref[:, 0:n_in])
                   + _dot_nk(h, wc_ref[:, n_in:]) + bc_ref[...])
          * jax.nn.sigmoid(_dot_nk(x, wi_ref[:, 0:n_in])
                           + _dot_nk(h, wi_ref[:, n_in:]) + bi_ref[...]))
    og = jax.nn.sigmoid(_dot_nk(x, wo_ref[:, 0:n_in])
                        + _dot_nk(h, wo_ref[:, n_in:]) + bo_ref[...])

    w = None
    acc = None
    for n in range(k):
        j = k - 1 - n
        c = cell_ref[j]
        if j >= 1:
            hc_ref[j - 1] = c
        if n == 0:
            w = -d
            acc = c * w
        else:
            w = w * ((float(n) - d) * (1.0 / (n + 1.0)))
            acc = acc + c * w
    cell = ci - acc
    hc_ref[k - 1] = cell
    hn = jnp.tanh(cell) * og
    hn_ref[...] = hn
    out_ref[...] = _dot_nk(hn, wout_ref[...]) + bout_ref[...]


def kernel(sample, hidden, cell_tensor, d_0, Wc, bc, Wi, bi, Wf, bf, Wo, bo,
           Wout, bout, *, interpret=False):
    k, b, h = cell_tensor.shape
    n_in = sample.shape[1]
    out_dim = Wout.shape[0]

    bm = 64
    full = lambda r, c: pl.BlockSpec((r, c), lambda i: (0, 0))
    row = lambda c: pl.BlockSpec((bm, c), lambda i: (i, 0))
    output, hidden_new, hc, d_values = pl.pallas_call(
        _mlstm_kernel,
        grid=(b // bm,),
        in_specs=[
            row(n_in),
            row(h),
            row(h),
            pl.BlockSpec((k, bm, h), lambda i: (0, i, 0)),
            full(h, n_in + h),
            full(h, n_in + h),
            full(h, n_in + h),
            full(h, n_in + 2 * h),
            full(out_dim, h),
            full(1, h),
            full(1, h),
            full(1, h),
            full(1, h),
            full(1, out_dim),
        ],
        out_specs=[
            row(out_dim),
            row(h),
            pl.BlockSpec((k, bm, h), lambda i: (0, i, 0)),
            row(h),
        ],
        out_shape=[
            jax.ShapeDtypeStruct((b, out_dim), jnp.float32),
            jax.ShapeDtypeStruct((b, h), jnp.float32),
            jax.ShapeDtypeStruct((k, b, h), jnp.float32),
            jax.ShapeDtypeStruct((b, h), jnp.float32),
        ],
        compiler_params=pltpu.CompilerParams(
            dimension_semantics=("parallel",),
            vmem_limit_bytes=56 * 1024 * 1024,
        ),
        name="mlstm_fused",
        interpret=interpret,
    )(sample, hidden, d_0, cell_tensor, Wi, Wo, Wc, Wf, Wout,
      bi.reshape(1, h), bo.reshape(1, h), bc.reshape(1, h), bf.reshape(1, h),
      bout.reshape(1, out_dim))

    return (output, hidden_new, hc, d_values)


# speedup vs baseline: 1.1426x; 1.1426x over previous
import jax
import jax.numpy as jnp
from jax.experimental import pallas as pl
from jax.experimental.pallas import tpu as pltpu


def _dot_nk(a, w_nk):
    return jax.lax.dot_general(
        a, w_nk, (((1,), (1,)), ((), ())),
        preferred_element_type=jnp.float32)


def _mlstm_kernel(x_ref, h_ref, d0_ref, cell_ref, wi_ref, wo_ref, wc_ref,
                  wf_ref, wout_ref, bi_ref, bo_ref, bc_ref, bf_ref, bout_ref,
                  out_ref, hn_ref, hc_ref, d_ref):
    n_in = x_ref.shape[1]
    k = cell_ref.shape[0]
    x = x_ref[...]
    h = h_ref[...]

    f_pre = (_dot_nk(x, wf_ref[:, 0:n_in])
             + _dot_nk(h, wf_ref[:, n_in:n_in + h.shape[1]])
             + _dot_nk(d0_ref[...], wf_ref[:, n_in + h.shape[1]:])
             + bf_ref[...])
    d = jax.nn.sigmoid(f_pre) * 0.5
    d_ref[...] = d
    ci = (jnp.tanh(_dot_nk(x, wc_ref[:, 0:n_in])
                   + _dot_nk(h, wc_ref[:, n_in:]) + bc_ref[...])
          * jax.nn.sigmoid(_dot_nk(x, wi_ref[:, 0:n_in])
                           + _dot_nk(h, wi_ref[:, n_in:]) + bi_ref[...]))
    og = jax.nn.sigmoid(_dot_nk(x, wo_ref[:, 0:n_in])
                        + _dot_nk(h, wo_ref[:, n_in:]) + bo_ref[...])

    w = None
    acc = None
    for n in range(k):
        j = k - 1 - n
        c = cell_ref[j]
        if j >= 1:
            hc_ref[j - 1] = c
        if n == 0:
            w = -d
            acc = c * w
        else:
            w = w * ((float(n) - d) * (1.0 / (n + 1.0)))
            acc = acc + c * w
    cell = ci - acc
    hc_ref[k - 1] = cell
    hn = jnp.tanh(cell) * og
    hn_ref[...] = hn
    out_ref[...] = _dot_nk(hn, wout_ref[...]) + bout_ref[...]


def kernel(sample, hidden, cell_tensor, d_0, Wc, bc, Wi, bi, Wf, bf, Wo, bo,
           Wout, bout, *, interpret=False):
    k, b, h = cell_tensor.shape
    n_in = sample.shape[1]
    out_dim = Wout.shape[0]

    bm = 128
    full = lambda r, c: pl.BlockSpec((r, c), lambda i: (0, 0))
    row = lambda c: pl.BlockSpec((bm, c), lambda i: (i, 0))
    output, hidden_new, hc, d_values = pl.pallas_call(
        _mlstm_kernel,
        grid=(b // bm,),
        in_specs=[
            row(n_in),
            row(h),
            row(h),
            pl.BlockSpec((k, bm, h), lambda i: (0, i, 0)),
            full(h, n_in + h),
            full(h, n_in + h),
            full(h, n_in + h),
            full(h, n_in + 2 * h),
            full(out_dim, h),
            full(1, h),
            full(1, h),
            full(1, h),
            full(1, h),
            full(1, out_dim),
        ],
        out_specs=[
            row(out_dim),
            row(h),
            pl.BlockSpec((k, bm, h), lambda i: (0, i, 0)),
            row(h),
        ],
        out_shape=[
            jax.ShapeDtypeStruct((b, out_dim), jnp.float32),
            jax.ShapeDtypeStruct((b, h), jnp.float32),
            jax.ShapeDtypeStruct((k, b, h), jnp.float32),
            jax.ShapeDtypeStruct((b, h), jnp.float32),
        ],
        compiler_params=pltpu.CompilerParams(
            dimension_semantics=("parallel",),
            vmem_limit_bytes=56 * 1024 * 1024,
        ),
        name="mlstm_fused",
        interpret=interpret,
    )(sample, hidden, d_0, cell_tensor, Wi, Wo, Wc, Wf, Wout,
      bi.reshape(1, h), bo.reshape(1, h), bc.reshape(1, h), bf.reshape(1, h),
      bout.reshape(1, out_dim))

    return (output, hidden_new, hc, d_values)
